# baseline (speedup 1.0000x reference)
import functools

import jax
import jax.numpy as jnp
from jax import lax
from jax.experimental import pallas as pl
from jax.experimental.pallas import tpu as pltpu
from jax.experimental.pallas import tpu_sc as plsc

B = 16384
F_SPARSE = 26
F_DENSE = 13
VOCAB = 1000000
L = 16
NC = 2
NS = 16
NW = NC * NS
ROWS = B // NW
NCH = ROWS // L


def _sc_body(xs_hbm, xd_hbm, table_hbm, wrep_hbm, out_hbm,
             xs_v, xd_v, wrep_v, idx_v, vals_v, acc_v, sem):
    wid = lax.axis_index("s") * NC + lax.axis_index("c")
    base = wid * ROWS

    pltpu.sync_copy(xs_hbm.at[:, pl.ds(base, ROWS)], xs_v)
    pltpu.sync_copy(xd_hbm.at[:, pl.ds(base, ROWS)], xd_v)
    pltpu.sync_copy(wrep_hbm, wrep_v)

    for f in range(F_SPARSE):
        off = f * VOCAB
        for j in range(NCH):
            sl = pl.ds(j * L, L)
            idx_v[pl.ds(f * ROWS + j * L, L)] = xs_v[f, sl] + off

    pltpu.async_copy(table_hbm.at[idx_v], vals_v, sem).wait()

    wk = [wrep_v[pl.ds(k * L, L)] for k in range(F_DENSE)]
    for j in range(NCH):
        sl = pl.ds(j * L, L)
        acc = vals_v[pl.ds(j * L, L)]
        for f in range(1, F_SPARSE):
            acc = acc + vals_v[pl.ds(f * ROWS + j * L, L)]
        for k in range(F_DENSE):
            acc = acc + xd_v[k, sl] * wk[k]
        acc_v[sl] = 1.0 / (1.0 + jnp.exp(-acc))

    pltpu.sync_copy(acc_v, out_hbm.at[pl.ds(base, ROWS)])


@jax.jit
def kernel(X_sparse, X_dense, lin_table, W):
    xs_t = X_sparse.T
    xd_t = X_dense.T
    table = lin_table.reshape(F_SPARSE * VOCAB)
    wrep = jnp.repeat(W.reshape(F_DENSE), L)

    mesh = plsc.VectorSubcoreMesh(core_axis_name="c", subcore_axis_name="s")
    run = pl.kernel(
        _sc_body,
        out_type=jax.ShapeDtypeStruct((B,), jnp.float32),
        mesh=mesh,
        scratch_types=[
            pltpu.VMEM((F_SPARSE, ROWS), jnp.int32),
            pltpu.VMEM((F_DENSE, ROWS), jnp.float32),
            pltpu.VMEM((F_DENSE * L,), jnp.float32),
            pltpu.VMEM((F_SPARSE * ROWS,), jnp.int32),
            pltpu.VMEM((F_SPARSE * ROWS,), jnp.float32),
            pltpu.VMEM((ROWS,), jnp.float32),
            pltpu.SemaphoreType.DMA,
        ],
    )
    out = run(xs_t, xd_t, table, wrep)
    return out.reshape(B, 1)


# speedup vs baseline: 18.2857x; 18.2857x over previous
import functools

import jax
import jax.numpy as jnp
from jax import lax
from jax.experimental import pallas as pl
from jax.experimental.pallas import tpu as pltpu
from jax.experimental.pallas import tpu_sc as plsc

B = 16384
F_SPARSE = 26
F_DENSE = 13
VOCAB = 1000000
VPAD = 1000448
L = 16
NC = 2
NS = 16
NW = NC * NS
ROWS = B // NW
NIDX = ROWS * F_SPARSE
NCH = ROWS // L


def _sc_body(xs_hbm, xd_hbm, table_hbm, wrep_hbm, out_hbm,
             xs_v, xd_v, wrep_v, idx_v, vals_v, acc_v, sem):
    wid = lax.axis_index("s") * NC + lax.axis_index("c")
    base = wid * ROWS

    pltpu.sync_copy(xs_hbm.at[:, pl.ds(base, ROWS)], xs_v)
    pltpu.sync_copy(xd_hbm.at[:, pl.ds(base, ROWS)], xd_v)
    pltpu.sync_copy(wrep_hbm, wrep_v)

    for f in range(F_SPARSE):
        off = f * VPAD
        for j in range(NCH):
            sl = pl.ds(j * L, L)
            idx_v[pl.ds(f * ROWS + j * L, L)] = xs_v[f, sl] + off

    pltpu.async_copy(table_hbm.at[idx_v], vals_v, sem).wait()

    wk = [wrep_v[pl.ds(k * L, L)] for k in range(F_DENSE)]
    for j in range(NCH):
        sl = pl.ds(j * L, L)
        acc = vals_v[pl.ds(j * L, L)]
        for f in range(1, F_SPARSE):
            acc = acc + vals_v[pl.ds(f * ROWS + j * L, L)]
        for k in range(F_DENSE):
            acc = acc + xd_v[k, sl] * wk[k]
        acc_v[sl] = 1.0 / (1.0 + jnp.exp(-acc))

    pltpu.sync_copy(acc_v, out_hbm.at[pl.ds(base, ROWS)])


@jax.jit
def kernel(X_sparse, X_dense, lin_table, W):
    xs_t = X_sparse.T
    xd_t = X_dense.T
    table = jnp.pad(lin_table, ((0, 0), (0, VPAD - VOCAB), (0, 0))).reshape(-1)
    wrep = jnp.repeat(W.reshape(F_DENSE), L)

    mesh = plsc.VectorSubcoreMesh(core_axis_name="c", subcore_axis_name="s")
    run = pl.kernel(
        _sc_body,
        out_type=jax.ShapeDtypeStruct((B,), jnp.float32),
        mesh=mesh,
        scratch_types=[
            pltpu.VMEM((F_SPARSE, ROWS), jnp.int32),
            pltpu.VMEM((F_DENSE, ROWS), jnp.float32),
            pltpu.VMEM((F_DENSE * L,), jnp.float32),
            pltpu.VMEM((NIDX,), jnp.int32),
            pltpu.VMEM((NIDX,), jnp.float32),
            pltpu.VMEM((ROWS,), jnp.float32),
            pltpu.SemaphoreType.DMA,
        ],
    )
    out = run(xs_t, xd_t, table, wrep)
    return out.reshape(B, 1)
